# Initial kernel scaffold; baseline (speedup 1.0000x reference)
import jax
import jax.numpy as jnp
from jax.experimental import pallas as pl


def kernel(x, ei, Ws1, Wn1, bl1, g1, be1, Ws2, Wn2, bl2, g2, be2, Ws3, Wn3, bl3, g3, be3, Wh, bh):
    raise NotImplementedError("write your pallas kernel here")


# speedup vs baseline: 4.5514x; 4.5514x over previous
import functools

import jax
import jax.numpy as jnp
from jax import lax
from jax.experimental import pallas as pl
from jax.experimental.pallas import tpu as pltpu
from jax.experimental.pallas import tpu_sc as plsc

N = 10000
E = 320000
D = 128

NC = 2
NS = 16
EPC = E // NC
EPT = EPC // NS
CH = 80
NCHUNK = EPT // CH
ROWS_A = 632
ROWS_LAST = N - (NS - 1) * ROWS_A

_MESH = plsc.VectorSubcoreMesh(core_axis_name="c", subcore_axis_name="s")


@functools.partial(
    pl.kernel,
    mesh=_MESH,
    out_type=jax.ShapeDtypeStruct((NC, N, D), jnp.float32),
    scratch_types=[
        pltpu.VMEM((CH,), jnp.int32),
        pltpu.VMEM((CH,), jnp.int32),
        pltpu.VMEM((CH, D), jnp.float32),
        pltpu.VMEM_SHARED((N, D), jnp.float32),
        pltpu.SemaphoreType.DMA,
    ],
)
def _segsum_sc(y_hbm, src_hbm, dst_hbm, zeros_hbm, out_hbm,
               sidx, didx, rows, acc, sem):
    cid = lax.axis_index("c")
    sid = lax.axis_index("s")
    roff = pl.multiple_of(sid * ROWS_A, 8)

    @pl.when(sid < NS - 1)
    def _():
        pltpu.sync_copy(zeros_hbm.at[pl.ds(roff, ROWS_A)],
                        acc.at[pl.ds(roff, ROWS_A)])

    @pl.when(sid == NS - 1)
    def _():
        pltpu.sync_copy(zeros_hbm.at[pl.ds(roff, ROWS_LAST)],
                        acc.at[pl.ds(roff, ROWS_LAST)])

    plsc.subcore_barrier()

    eb = cid * EPC + sid * EPT

    def body(j, carry):
        off = pl.multiple_of(eb + j * CH, 8)
        pltpu.sync_copy(src_hbm.at[pl.ds(off, CH)], sidx)
        pltpu.sync_copy(dst_hbm.at[pl.ds(off, CH)], didx)
        pltpu.async_copy(y_hbm.at[sidx], rows, sem).wait()
        pltpu.sync_copy(rows, acc.at[didx], add=True)
        return carry

    lax.fori_loop(0, NCHUNK, body, 0)

    plsc.subcore_barrier()

    @pl.when(sid < NS - 1)
    def _():
        pltpu.sync_copy(acc.at[pl.ds(roff, ROWS_A)],
                        out_hbm.at[cid, pl.ds(roff, ROWS_A)])

    @pl.when(sid == NS - 1)
    def _():
        pltpu.sync_copy(acc.at[pl.ds(roff, ROWS_LAST)],
                        out_hbm.at[cid, pl.ds(roff, ROWS_LAST)])


def _mmb(a, b):
    return jnp.dot(a.astype(jnp.bfloat16), b.astype(jnp.bfloat16),
                   preferred_element_type=jnp.float32)


def _bn_relu(z, g, be):
    m = jnp.mean(z, axis=0, keepdims=True)
    zc = z - m
    v = jnp.mean(zc * zc, axis=0, keepdims=True)
    return jnp.maximum(g * zc * lax.rsqrt(v + 1e-5) + be, 0.0)


def _combine_body(h_ref, p_ref, ws_ref, wn_ref, bl_ref, g_ref, be_ref,
                  ho_ref):
    agg = p_ref[0] + p_ref[1]
    z = _mmb(h_ref[...], ws_ref[...]) + _mmb(agg, wn_ref[...]) + bl_ref[...]
    ho_ref[...] = _bn_relu(z, g_ref[...], be_ref[...])


def _combine(h, p, ws, wn, bl, g, be):
    return pl.pallas_call(
        _combine_body,
        out_shape=jax.ShapeDtypeStruct((N, D), jnp.float32),
    )(h, p, ws, wn, bl.reshape(1, D), g.reshape(1, D), be.reshape(1, D))


def _head_body(h_ref, p_ref, ws_ref, wn_ref, bl_ref, g_ref, be_ref,
               wh_ref, bh_ref, o_ref):
    agg = p_ref[0] + p_ref[1]
    z = _mmb(h_ref[...], ws_ref[...]) + _mmb(agg, wn_ref[...]) + bl_ref[...]
    hn = _bn_relu(z, g_ref[...], be_ref[...])
    o_ref[...] = _mmb(hn, wh_ref[...]) + bh_ref[...]


def _head(h, p, ws, wn, bl, g, be, wh, bh):
    return pl.pallas_call(
        _head_body,
        out_shape=jax.ShapeDtypeStruct((N, D), jnp.float32),
    )(h, p, ws, wn, bl.reshape(1, D), g.reshape(1, D), be.reshape(1, D), wh,
      bh.reshape(1, D))


def kernel(x, ei, Ws1, Wn1, bl1, g1, be1, Ws2, Wn2, bl2, g2, be2,
           Ws3, Wn3, bl3, g3, be3, Wh, bh):
    src = ei[0]
    dst = ei[1]
    zeros = jnp.zeros((N, D), jnp.float32)

    p1 = _segsum_sc(x, src, dst, zeros)
    h1 = _combine(x, p1, Ws1, Wn1, bl1, g1, be1)
    p2 = _segsum_sc(h1, src, dst, zeros)
    h2 = _combine(h1, p2, Ws2, Wn2, bl2, g2, be2)
    p3 = _segsum_sc(h2, src, dst, zeros)
    return _head(h2, p3, Ws3, Wn3, bl3, g3, be3, Wh, bh)


# speedup vs baseline: 11.5250x; 2.5322x over previous
import functools

import jax
import jax.numpy as jnp
from jax import lax
from jax.experimental import pallas as pl
from jax.experimental.pallas import tpu as pltpu
from jax.experimental.pallas import tpu_sc as plsc

N = 10000
E = 320000
D = 128

NC = 2
NS = 16
EPC = E // NC
EPT = EPC // NS
CH = 128
NFULL = EPT // CH
TAIL = EPT - NFULL * CH
ROWS_A = 632
ROWS_LAST = N - (NS - 1) * ROWS_A

_MESH = plsc.VectorSubcoreMesh(core_axis_name="c", subcore_axis_name="s")


@functools.partial(
    pl.kernel,
    mesh=_MESH,
    out_type=jax.ShapeDtypeStruct((NC, N, D), jnp.float32),
    scratch_types=[
        pltpu.VMEM((EPT,), jnp.int32),
        pltpu.VMEM((CH,), jnp.int32),
        pltpu.VMEM((CH,), jnp.int32),
        pltpu.VMEM((CH, D), jnp.float32),
        pltpu.VMEM((CH, D), jnp.float32),
        pltpu.VMEM((TAIL,), jnp.int32),
        pltpu.VMEM((TAIL, D), jnp.float32),
        pltpu.VMEM_SHARED((N, D), jnp.float32),
        pltpu.SemaphoreType.DMA,
        pltpu.SemaphoreType.DMA,
    ],
)
def _segsum_sc(y_hbm, src_hbm, dst_hbm, zeros_hbm, out_hbm,
               sidx_all, didx0, didx1, rows0, rows1,
               didx_t, rows_t, acc, sem0, sem1):
    cid = lax.axis_index("c")
    sid = lax.axis_index("s")
    roff = pl.multiple_of(sid * ROWS_A, 8)

    @pl.when(sid < NS - 1)
    def _():
        pltpu.sync_copy(zeros_hbm.at[pl.ds(roff, ROWS_A)],
                        acc.at[pl.ds(roff, ROWS_A)])

    @pl.when(sid == NS - 1)
    def _():
        pltpu.sync_copy(zeros_hbm.at[pl.ds(roff, ROWS_LAST)],
                        acc.at[pl.ds(roff, ROWS_LAST)])

    eb = pl.multiple_of((cid * NS + sid) * EPT, 8)
    pltpu.sync_copy(src_hbm.at[pl.ds(eb, EPT)], sidx_all)
    plsc.subcore_barrier()

    bufs = ((didx0, rows0, sem0), (didx1, rows1, sem1))

    def start(jj, b):
        didx, rows, sem = bufs[b]
        lo = pl.multiple_of(jj * CH, 8)
        pltpu.async_copy(dst_hbm.at[pl.ds(eb + lo, CH)], didx, sem)
        pltpu.async_copy(y_hbm.at[sidx_all.at[pl.ds(lo, CH)]], rows, sem)

    start(0, 0)
    start(1, 1)

    def body(k, carry):
        for b in (0, 1):
            jj = 2 * k + b
            didx, rows, sem = bufs[b]
            pltpu.make_async_copy(dst_hbm.at[pl.ds(0, CH)], didx, sem).wait()
            pltpu.make_async_copy(y_hbm.at[pl.ds(0, CH)], rows, sem).wait()
            pltpu.sync_copy(rows, acc.at[didx], add=True)
            nxt = jj + 2

            @pl.when(nxt < NFULL)
            def _():
                start(nxt, b)

        return carry

    lax.fori_loop(0, NFULL // 2, body, 0)

    lo_t = pl.multiple_of(NFULL * CH, 8)
    pltpu.sync_copy(dst_hbm.at[pl.ds(eb + lo_t, TAIL)], didx_t)
    pltpu.async_copy(y_hbm.at[sidx_all.at[pl.ds(lo_t, TAIL)]],
                     rows_t, sem0).wait()
    pltpu.sync_copy(rows_t, acc.at[didx_t], add=True)

    plsc.subcore_barrier()

    @pl.when(sid < NS - 1)
    def _():
        pltpu.sync_copy(acc.at[pl.ds(roff, ROWS_A)],
                        out_hbm.at[cid, pl.ds(roff, ROWS_A)])

    @pl.when(sid == NS - 1)
    def _():
        pltpu.sync_copy(acc.at[pl.ds(roff, ROWS_LAST)],
                        out_hbm.at[cid, pl.ds(roff, ROWS_LAST)])


def _mmb(a, b):
    return jnp.dot(a.astype(jnp.bfloat16), b.astype(jnp.bfloat16),
                   preferred_element_type=jnp.float32)


def _bn_relu(z, g, be):
    m = jnp.mean(z, axis=0, keepdims=True)
    zc = z - m
    v = jnp.mean(zc * zc, axis=0, keepdims=True)
    return jnp.maximum(g * zc * lax.rsqrt(v + 1e-5) + be, 0.0)


def _combine_body(h_ref, p_ref, ws_ref, wn_ref, bl_ref, g_ref, be_ref,
                  ho_ref):
    agg = p_ref[0] + p_ref[1]
    z = _mmb(h_ref[...], ws_ref[...]) + _mmb(agg, wn_ref[...]) + bl_ref[...]
    ho_ref[...] = _bn_relu(z, g_ref[...], be_ref[...])


def _combine(h, p, ws, wn, bl, g, be):
    return pl.pallas_call(
        _combine_body,
        out_shape=jax.ShapeDtypeStruct((N, D), jnp.float32),
    )(h, p, ws, wn, bl.reshape(1, D), g.reshape(1, D), be.reshape(1, D))


def _head_body(h_ref, p_ref, ws_ref, wn_ref, bl_ref, g_ref, be_ref,
               wh_ref, bh_ref, o_ref):
    agg = p_ref[0] + p_ref[1]
    z = _mmb(h_ref[...], ws_ref[...]) + _mmb(agg, wn_ref[...]) + bl_ref[...]
    hn = _bn_relu(z, g_ref[...], be_ref[...])
    o_ref[...] = _mmb(hn, wh_ref[...]) + bh_ref[...]


def _head(h, p, ws, wn, bl, g, be, wh, bh):
    return pl.pallas_call(
        _head_body,
        out_shape=jax.ShapeDtypeStruct((N, D), jnp.float32),
    )(h, p, ws, wn, bl.reshape(1, D), g.reshape(1, D), be.reshape(1, D), wh,
      bh.reshape(1, D))


def kernel(x, ei, Ws1, Wn1, bl1, g1, be1, Ws2, Wn2, bl2, g2, be2,
           Ws3, Wn3, bl3, g3, be3, Wh, bh):
    src = ei[0]
    dst = ei[1]
    zeros = jnp.zeros((N, D), jnp.float32)

    p1 = _segsum_sc(x, src, dst, zeros)
    h1 = _combine(x, p1, Ws1, Wn1, bl1, g1, be1)
    p2 = _segsum_sc(h1, src, dst, zeros)
    h2 = _combine(h1, p2, Ws2, Wn2, bl2, g2, be2)
    p3 = _segsum_sc(h2, src, dst, zeros)
    return _head(h2, p3, Ws3, Wn3, bl3, g3, be3, Wh, bh)
